# baseline (speedup 1.0000x reference)
import jax
import jax.numpy as jnp
from jax import lax
from jax.experimental import pallas as pl
from jax.experimental.pallas import tpu as pltpu
from jax.experimental.pallas import tpu_sc as plsc

BATCH = 16384
EMBED_DIM = 64
LANES = 16
CHUNK = 128

_info = plsc.get_sparse_core_info()
NC, NS = _info.num_cores, _info.num_subcores
NW = NC * NS
B_PER_W = BATCH // NW
N_CHUNKS = B_PER_W // CHUNK


def _sc_body(uids, iids, utab, itab, w_hbm, b_hbm, out_hbm,
             uidx_v, iidx_v, urows_v, irows_v, w_v, b_v, out_v, sem_u, sem_i):
    wid = lax.axis_index("s") * NC + lax.axis_index("c")
    base = wid * B_PER_W

    pltpu.sync_copy(uids.at[pl.ds(wid * N_CHUNKS, N_CHUNKS)], uidx_v)
    pltpu.sync_copy(iids.at[pl.ds(wid * N_CHUNKS, N_CHUNKS)], iidx_v)
    pltpu.sync_copy(w_hbm, w_v)
    pltpu.sync_copy(b_hbm, b_v)

    copies = []
    for j in range(N_CHUNKS):
        dst = pl.ds(j * CHUNK, CHUNK)
        copies.append(pltpu.async_copy(utab.at[uidx_v.at[j]], urows_v.at[dst], sem_u))
        copies.append(pltpu.async_copy(itab.at[iidx_v.at[j]], irows_v.at[dst], sem_i))
    for c in copies:
        c.wait()

    w_chunks = [w_v[pl.ds(k * LANES, LANES)] for k in range(EMBED_DIM // LANES)]
    bvec = b_v[pl.ds(0, LANES)]
    iota16 = lax.iota(jnp.int32, LANES)
    cols = [jnp.full((LANES,), d, jnp.int32) for d in range(EMBED_DIM)]

    def group(g, carry):
        rows = g * LANES + iota16
        acc = bvec
        for d in range(EMBED_DIM):
            ug = plsc.load_gather(urows_v, [rows, cols[d]])
            ig = plsc.load_gather(irows_v, [rows, cols[d]])
            acc = acc + ug * ig * w_chunks[d // LANES][d % LANES]
        out_v[pl.ds(g * LANES, LANES)] = acc
        return carry

    lax.fori_loop(0, B_PER_W // LANES, group, 0)

    pltpu.sync_copy(out_v, out_hbm.at[pl.ds(base, B_PER_W)])


@jax.jit
def _run(uids2d, iids2d, user_table, item_table, w_vec, b_vec):
    mesh = plsc.VectorSubcoreMesh(core_axis_name="c", subcore_axis_name="s")
    return pl.kernel(
        _sc_body,
        out_type=jax.ShapeDtypeStruct((BATCH,), jnp.float32),
        mesh=mesh,
        compiler_params=pltpu.CompilerParams(
            needs_layout_passes=False, use_tc_tiling_on_sc=False),
        scratch_types=[
            pltpu.VMEM((N_CHUNKS, CHUNK), jnp.int32),
            pltpu.VMEM((N_CHUNKS, CHUNK), jnp.int32),
            pltpu.VMEM((B_PER_W, EMBED_DIM), jnp.float32),
            pltpu.VMEM((B_PER_W, EMBED_DIM), jnp.float32),
            pltpu.VMEM((EMBED_DIM,), jnp.float32),
            pltpu.VMEM((LANES,), jnp.float32),
            pltpu.VMEM((B_PER_W,), jnp.float32),
            pltpu.SemaphoreType.DMA,
            pltpu.SemaphoreType.DMA,
        ],
    )(uids2d, iids2d, user_table, item_table, w_vec, b_vec)


def kernel(user_ids, item_ids, user_table, item_table, fc_w, fc_b):
    uids2d = user_ids.reshape(BATCH // CHUNK, CHUNK)
    iids2d = item_ids.reshape(BATCH // CHUNK, CHUNK)
    w_vec = fc_w.reshape(EMBED_DIM)
    b_vec = jnp.broadcast_to(fc_b, (LANES,))
    return _run(uids2d, iids2d, user_table, item_table, w_vec, b_vec)


# speedup vs baseline: 1.6896x; 1.6896x over previous
import jax
import jax.numpy as jnp
from jax import lax
from jax.experimental import pallas as pl
from jax.experimental.pallas import tpu as pltpu
from jax.experimental.pallas import tpu_sc as plsc

BATCH = 16384
TBLK = 2048
EMBED_DIM = 64
LANES = 16
IDX_CHUNK = 128
PAIR_W = 2 * EMBED_DIM

_info = plsc.get_sparse_core_info()
NC, NS = _info.num_cores, _info.num_subcores
NW = NC * NS
B_PER_W = BATCH // NW
N_CHUNKS = B_PER_W // IDX_CHUNK
N_PASS = 2
P_ROWS = B_PER_W // N_PASS
CH_PER_PASS = N_CHUNKS // N_PASS


def _sc_body(upair, upar, ipair, ipar, utab2, itab2, w_hbm, b_hbm, out_hbm,
             uidx_v, iidx_v, upar_v, ipar_v, urows_v, irows_v, w_v, b_v,
             out_v, sem):
    wid = lax.axis_index("s") * NC + lax.axis_index("c")
    base = wid * B_PER_W

    for k in range(N_CHUNKS):
        sl = pl.ds(base + k * IDX_CHUNK, IDX_CHUNK)
        pltpu.sync_copy(upair.at[sl], uidx_v.at[k])
        pltpu.sync_copy(ipair.at[sl], iidx_v.at[k])
    pltpu.sync_copy(upar.at[pl.ds(base, B_PER_W)], upar_v)
    pltpu.sync_copy(ipar.at[pl.ds(base, B_PER_W)], ipar_v)
    pltpu.sync_copy(w_hbm, w_v)
    pltpu.sync_copy(b_hbm, b_v)

    w_chunks = [w_v[pl.ds(k * LANES, LANES)] for k in range(EMBED_DIM // LANES)]
    bias = b_v[pl.ds(0, LANES)][0]
    iota16 = lax.iota(jnp.int32, LANES)

    for t in range(N_PASS):
        copies = []
        for k in range(CH_PER_PASS):
            dst = pl.ds(k * IDX_CHUNK, IDX_CHUNK)
            copies.append(pltpu.async_copy(
                utab2.at[uidx_v.at[t * CH_PER_PASS + k]], urows_v.at[dst], sem))
            copies.append(pltpu.async_copy(
                itab2.at[iidx_v.at[t * CH_PER_PASS + k]], irows_v.at[dst], sem))
        for c in copies:
            c.wait()

        def group(g, carry):
            rbase = g * LANES
            pu16 = upar_v[pl.ds(t * P_ROWS + rbase, LANES)]
            pi16 = ipar_v[pl.ds(t * P_ROWS + rbase, LANES)]
            out16 = jnp.zeros((LANES,), jnp.float32)
            for j in range(LANES):
                r = rbase + j
                ou = pu16[j] * EMBED_DIM
                oi = pi16[j] * EMBED_DIM
                acc = None
                for k in range(EMBED_DIM // LANES):
                    u = urows_v[r, pl.ds(ou + k * LANES, LANES)]
                    i = irows_v[r, pl.ds(oi + k * LANES, LANES)]
                    term = u * i * w_chunks[k]
                    acc = term if acc is None else acc + term
                s = jnp.sum(acc) + bias
                out16 = jnp.where(iota16 == j, s, out16)
            out_v[pl.ds(t * P_ROWS + rbase, LANES)] = out16
            return carry

        lax.fori_loop(0, P_ROWS // LANES, group, 0)

    pltpu.sync_copy(out_v, out_hbm.at[pl.ds(base, B_PER_W)])


@jax.jit
def _run(upair, upar, ipair, ipar, utab2, itab2, w_vec, b_vec):
    mesh = plsc.VectorSubcoreMesh(core_axis_name="c", subcore_axis_name="s")
    return pl.kernel(
        _sc_body,
        out_type=jax.ShapeDtypeStruct((BATCH,), jnp.float32),
        mesh=mesh,
        compiler_params=pltpu.CompilerParams(
            needs_layout_passes=False, use_tc_tiling_on_sc=True),
        scratch_types=[
            pltpu.VMEM((N_CHUNKS, IDX_CHUNK), jnp.int32),
            pltpu.VMEM((N_CHUNKS, IDX_CHUNK), jnp.int32),
            pltpu.VMEM((B_PER_W,), jnp.int32),
            pltpu.VMEM((B_PER_W,), jnp.int32),
            pltpu.VMEM((P_ROWS, PAIR_W), jnp.float32),
            pltpu.VMEM((P_ROWS, PAIR_W), jnp.float32),
            pltpu.VMEM((EMBED_DIM,), jnp.float32),
            pltpu.VMEM((LANES,), jnp.float32),
            pltpu.VMEM((B_PER_W,), jnp.float32),
            pltpu.SemaphoreType.DMA,
        ],
    )(upair, upar, ipair, ipar, utab2, itab2, w_vec, b_vec)


def _tc_pack_body(ut_ref, it_ref, uo_ref, io_ref):
    half = TBLK // 2
    for src, dst in ((ut_ref, uo_ref), (it_ref, io_ref)):
        xt = jnp.transpose(src[...])
        dst[:, 0:EMBED_DIM] = xt[0:half]
        dst[:, EMBED_DIM:PAIR_W] = xt[half:TBLK]


@jax.jit
def _pack_tables(utab_t, itab_t):
    n = utab_t.shape[1]
    grid = (n + TBLK - 1) // TBLK
    in_spec = pl.BlockSpec((EMBED_DIM, TBLK), lambda j: (0, j))
    out_spec = pl.BlockSpec((TBLK // 2, PAIR_W), lambda j: (j, 0))
    out_sds = jax.ShapeDtypeStruct((grid * (TBLK // 2), PAIR_W), jnp.float32)
    return pl.pallas_call(
        _tc_pack_body,
        grid=(grid,),
        in_specs=[in_spec, in_spec],
        out_specs=[out_spec, out_spec],
        out_shape=[out_sds, out_sds],
    )(utab_t, itab_t)


def kernel(user_ids, item_ids, user_table, item_table, fc_w, fc_b):
    utab2, itab2 = _pack_tables(user_table.T, item_table.T)
    upair = ((user_ids >> 11) << 10) | (user_ids & 1023)
    upar = (user_ids >> 10) & 1
    ipair = ((item_ids >> 11) << 10) | (item_ids & 1023)
    ipar = (item_ids >> 10) & 1
    w_vec = fc_w.reshape(EMBED_DIM)
    b_vec = jnp.broadcast_to(fc_b, (LANES,))
    return _run(upair, upar, ipair, ipar, utab2, itab2, w_vec, b_vec)
